# Initial kernel scaffold; baseline (speedup 1.0000x reference)
import jax
import jax.numpy as jnp
from jax.experimental import pallas as pl


def kernel(xyz, new_xyz, features):
    raise NotImplementedError("write your pallas kernel here")


# speedup vs baseline: 9.7310x; 9.7310x over previous
import functools

import jax
import jax.numpy as jnp
from jax import lax
from jax.experimental import pallas as pl
from jax.experimental.pallas import tpu as pltpu
from jax.experimental.pallas import tpu_sc as plsc

B, N, P, S, C = 4, 8192, 2048, 32, 256
CT = C + 3
R_IN2 = 0.8 * 0.8
R_OUT2 = 1.6 * 1.6
NC, NS, L = 2, 16, 16
NW = NC * NS
WPB = NW // B
QPW = P // WPB
SEG = 16384


def _worker_id():
    return lax.axis_index("s") * NC + lax.axis_index("c")


def _ballq_body(xyz_f, new_xyz_f, idx_out, cnt_out,
                xv, yv, zv, qx, qy, qz, idx_st, cnt_st, buf):
    w = _worker_id()
    b = w // WPB
    wi = w % WPB
    q0 = wi * QPW

    pltpu.sync_copy(xyz_f.at[pl.ds((b * 3 + 0) * N, N)], xv)
    pltpu.sync_copy(xyz_f.at[pl.ds((b * 3 + 1) * N, N)], yv)
    pltpu.sync_copy(xyz_f.at[pl.ds((b * 3 + 2) * N, N)], zv)
    pltpu.sync_copy(new_xyz_f.at[pl.ds((b * 3 + 0) * P + q0, QPW)], qx)
    pltpu.sync_copy(new_xyz_f.at[pl.ds((b * 3 + 1) * P + q0, QPW)], qy)
    pltpu.sync_copy(new_xyz_f.at[pl.ds((b * 3 + 2) * P + q0, QPW)], qz)

    iota = lax.iota(jnp.int32, L)
    zeros = jnp.zeros((L,), jnp.int32)
    lane0 = iota == 0

    def per_query(i, carry):
        fi = jnp.full((L,), i, jnp.int32)
        qxb = plsc.load_gather(qx, [fi])
        qyb = plsc.load_gather(qy, [fi])
        qzb = plsc.load_gather(qz, [fi])

        def cond(st):
            n, cnt = st
            return jnp.logical_and(cnt < S, n < N)

        def body(st):
            n, cnt = st
            dx = xv[pl.ds(n, L)] - qxb
            dy = yv[pl.ds(n, L)] - qyb
            dz = zv[pl.ds(n, L)] - qzb
            d2 = dx * dx + dy * dy + dz * dz
            m = jnp.logical_and(d2 >= R_IN2, d2 < R_OUT2)
            mi = m.astype(jnp.int32)
            tot = jnp.sum(mi)
            csum = plsc.cumsum(mi)
            pos = (cnt + csum) - mi
            plsc.store_scatter(buf, [pos], n + iota, mask=m)
            return n + L, cnt + tot

        _, cnt_end = lax.while_loop(cond, body, (jnp.int32(0), jnp.int32(0)))
        cntc = jnp.minimum(cnt_end, S)
        cntv = jnp.full((L,), cntc, jnp.int32)
        sv0 = buf[pl.ds(0, L)]
        h0 = jnp.where(cntc > 0, sv0[0], 0)
        first = jnp.full((L,), h0, jnp.int32)
        for j in range(S // L):
            sv = sv0 if j == 0 else buf[pl.ds(j * L, L)]
            outv = jnp.where(iota + (j * L) < cntv, sv, first)
            idx_st[pl.ds(i * S + j * L, L)] = outv
        plsc.store_scatter(cnt_st, [fi], cntv, mask=lane0)
        return carry

    lax.fori_loop(0, QPW, per_query, 0)
    pltpu.sync_copy(idx_st, idx_out.at[pl.ds(b * P * S + q0 * S, QPW * S)])
    pltpu.sync_copy(cnt_st, cnt_out.at[pl.ds(b * P + q0, QPW)])


def _gather_body(xyz_f, new_xyz_f, feat_f, idx_f, out,
                 idxv, src, qrow, obuf):
    w = _worker_id()
    b = w // WPB
    wi = w % WPB
    r0 = wi * 32 + jnp.minimum(wi, 3)
    nr = jnp.where(wi < 3, 33, 32)

    pltpu.sync_copy(idx_f.at[pl.ds(b * P * S, P * S)], idxv)
    iota = lax.iota(jnp.int32, L)

    def per_row(r, carry):
        cc = r0 + r

        def load_xyz():
            pltpu.sync_copy(xyz_f.at[pl.ds((b * 3 + cc) * N, N)], src)
            pltpu.sync_copy(new_xyz_f.at[pl.ds((b * 3 + cc) * P, P)], qrow)

        def load_feat():
            pltpu.sync_copy(feat_f.at[pl.ds((b * C + cc - 3) * N, N)], src)

        lax.cond(cc < 3, load_xyz, load_feat)
        obase = (b * CT + cc) * (P * S)

        def seg_loop(is_xyz):
            def per_seg(g, c2):
                base = g * SEG

                def per_chunk(t, c3):
                    off = t * L
                    iv = idxv[pl.ds(base + off, L)]
                    val = plsc.load_gather(src, [iv])
                    if is_xyz:
                        pvec = lax.shift_right_logical(base + off + iota, 5)
                        val = val - plsc.load_gather(qrow, [pvec])
                    obuf[pl.ds(off, L)] = val
                    return c3

                lax.fori_loop(0, SEG // L, per_chunk, 0)
                pltpu.sync_copy(obuf, out.at[pl.ds(obase + base, SEG)])
                return c2

            lax.fori_loop(0, (P * S) // SEG, per_seg, 0)

        lax.cond(cc < 3, lambda: seg_loop(True), lambda: seg_loop(False))
        return carry

    lax.fori_loop(0, nr, per_row, 0)


def kernel(xyz, new_xyz, features):
    xyz_f = jnp.transpose(xyz, (0, 2, 1)).reshape(-1)
    new_xyz_f = jnp.transpose(new_xyz, (0, 2, 1)).reshape(-1)
    feat_f = features.reshape(-1)

    mesh = plsc.VectorSubcoreMesh(
        core_axis_name="c", subcore_axis_name="s",
        num_cores=NC, num_subcores=NS)
    cparams = pltpu.CompilerParams(needs_layout_passes=False)

    ballq = functools.partial(
        pl.kernel,
        compiler_params=cparams,
        out_type=(
            jax.ShapeDtypeStruct((B * P * S,), jnp.int32),
            jax.ShapeDtypeStruct((B * P,), jnp.int32),
        ),
        mesh=mesh,
        scratch_types=[
            pltpu.VMEM((N,), jnp.float32),
            pltpu.VMEM((N,), jnp.float32),
            pltpu.VMEM((N,), jnp.float32),
            pltpu.VMEM((QPW,), jnp.float32),
            pltpu.VMEM((QPW,), jnp.float32),
            pltpu.VMEM((QPW,), jnp.float32),
            pltpu.VMEM((QPW * S,), jnp.int32),
            pltpu.VMEM((QPW,), jnp.int32),
            pltpu.VMEM((64,), jnp.int32),
        ],
    )(_ballq_body)
    idx_flat, idx_cnt = ballq(xyz_f, new_xyz_f)

    gath = functools.partial(
        pl.kernel,
        compiler_params=cparams,
        out_type=jax.ShapeDtypeStruct((B * CT * P * S,), jnp.float32),
        mesh=mesh,
        scratch_types=[
            pltpu.VMEM((P * S,), jnp.int32),
            pltpu.VMEM((N,), jnp.float32),
            pltpu.VMEM((P,), jnp.float32),
            pltpu.VMEM((SEG,), jnp.float32),
        ],
    )(_gather_body)
    grouped = gath(xyz_f, new_xyz_f, feat_f, idx_flat)

    return idx_cnt.reshape(B, P), grouped.reshape(B, CT, P, S)


# speedup vs baseline: 15.9905x; 1.6433x over previous
import functools

import jax
import jax.numpy as jnp
from jax import lax
from jax.experimental import pallas as pl
from jax.experimental.pallas import tpu as pltpu
from jax.experimental.pallas import tpu_sc as plsc

B, N, P, S, C = 4, 8192, 2048, 32, 256
CT = C + 3
R_IN2 = 0.8 * 0.8
R_OUT2 = 1.6 * 1.6
NC, NS, L = 2, 16, 16
NW = NC * NS
WPB = NW // B
QPW = P // WPB
SSEG = 8


def _worker_id():
    return lax.axis_index("s") * NC + lax.axis_index("c")


def _ballq_body(xyz_f, new_xyz_f, idx_out, cnt_out,
                xv, yv, zv, qx, qy, qz, idx_st, cnt_st, buf):
    w = _worker_id()
    b = w // WPB
    wi = w % WPB
    q0 = wi * QPW

    pltpu.sync_copy(xyz_f.at[pl.ds((b * 3 + 0) * N, N)], xv)
    pltpu.sync_copy(xyz_f.at[pl.ds((b * 3 + 1) * N, N)], yv)
    pltpu.sync_copy(xyz_f.at[pl.ds((b * 3 + 2) * N, N)], zv)
    pltpu.sync_copy(new_xyz_f.at[pl.ds((b * 3 + 0) * P + q0, QPW)], qx)
    pltpu.sync_copy(new_xyz_f.at[pl.ds((b * 3 + 1) * P + q0, QPW)], qy)
    pltpu.sync_copy(new_xyz_f.at[pl.ds((b * 3 + 2) * P + q0, QPW)], qz)

    iota = lax.iota(jnp.int32, L)
    lane0 = iota == 0

    def per_query(i, carry):
        fi = jnp.full((L,), i, jnp.int32)
        qxb = plsc.load_gather(qx, [fi])
        qyb = plsc.load_gather(qy, [fi])
        qzb = plsc.load_gather(qz, [fi])

        def cond(st):
            n, cnt = st
            return jnp.logical_and(cnt < S, n < N)

        def body(st):
            n, cnt = st
            dx = xv[pl.ds(n, L)] - qxb
            dy = yv[pl.ds(n, L)] - qyb
            dz = zv[pl.ds(n, L)] - qzb
            d2 = dx * dx + dy * dy + dz * dz
            m = jnp.logical_and(d2 >= R_IN2, d2 < R_OUT2)
            mi = m.astype(jnp.int32)
            tot = jnp.sum(mi)
            csum = plsc.cumsum(mi)
            pos = (cnt + csum) - mi
            plsc.store_scatter(buf, [pos], n + iota, mask=m)
            return n + L, cnt + tot

        _, cnt_end = lax.while_loop(cond, body, (jnp.int32(0), jnp.int32(0)))
        cntc = jnp.minimum(cnt_end, S)
        cntv = jnp.full((L,), cntc, jnp.int32)
        sv0 = buf[pl.ds(0, L)]
        h0 = jnp.where(cntc > 0, sv0[0], 0)
        first = jnp.full((L,), h0, jnp.int32)
        for j in range(S // L):
            sv = sv0 if j == 0 else buf[pl.ds(j * L, L)]
            outv = jnp.where(iota + (j * L) < cntv, sv, first)
            plsc.store_scatter(idx_st, [iota + (j * L), fi], outv)
        plsc.store_scatter(cnt_st, [fi], cntv, mask=lane0)
        return carry

    lax.fori_loop(0, QPW, per_query, 0)
    pltpu.sync_copy(idx_st, idx_out.at[pl.ds(b * S, S), pl.ds(q0, QPW)])
    pltpu.sync_copy(cnt_st, cnt_out.at[pl.ds(b * P + q0, QPW)])


def _gather_body(xyz_f, new_xyz_f, feat_f, idxT, out4,
                 idxv, src, qrow, obuf):
    w = _worker_id()
    b = w // WPB
    wi = w % WPB
    r0 = wi * 32 + jnp.minimum(wi, 3)
    nr = jnp.where(wi < 3, 33, 32)

    pltpu.sync_copy(idxT.at[pl.ds(b * S, S), :], idxv)

    def per_row(r, carry):
        cc = r0 + r

        def load_xyz():
            pltpu.sync_copy(xyz_f.at[pl.ds((b * 3 + cc) * N, N)], src)
            pltpu.sync_copy(new_xyz_f.at[pl.ds((b * 3 + cc) * P, P)], qrow)

        def load_feat():
            pltpu.sync_copy(feat_f.at[pl.ds((b * C + cc - 3) * N, N)], src)

        lax.cond(cc < 3, load_xyz, load_feat)

        def seg_loop(is_xyz):
            def per_seg(g, c2):
                def per_srow(sl, c3):
                    s = g * SSEG + sl

                    def per_pc(t, c4):
                        for u in range(4):
                            off = t * (4 * L) + u * L
                            iv = idxv[s, pl.ds(off, L)]
                            val = plsc.load_gather(src, [iv])
                            if is_xyz:
                                val = val - qrow[pl.ds(off, L)]
                            obuf[0, 0, sl, pl.ds(off, L)] = val
                        return c4

                    lax.fori_loop(0, P // (4 * L), per_pc, 0)
                    return c3

                lax.fori_loop(0, SSEG, per_srow, 0)
                pltpu.sync_copy(
                    obuf,
                    out4.at[pl.ds(b, 1), pl.ds(cc, 1), pl.ds(g * SSEG, SSEG), :])
                return c2

            lax.fori_loop(0, S // SSEG, per_seg, 0)

        lax.cond(cc < 3, lambda: seg_loop(True), lambda: seg_loop(False))
        return carry

    lax.fori_loop(0, nr, per_row, 0)


def kernel(xyz, new_xyz, features):
    xyz_f = jnp.transpose(xyz, (0, 2, 1)).reshape(-1)
    new_xyz_f = jnp.transpose(new_xyz, (0, 2, 1)).reshape(-1)
    feat_f = features.reshape(-1)

    mesh = plsc.VectorSubcoreMesh(
        core_axis_name="c", subcore_axis_name="s",
        num_cores=NC, num_subcores=NS)
    cparams = pltpu.CompilerParams(needs_layout_passes=False)

    ballq = functools.partial(
        pl.kernel,
        compiler_params=cparams,
        out_type=(
            jax.ShapeDtypeStruct((B * S, P), jnp.int32),
            jax.ShapeDtypeStruct((B * P,), jnp.int32),
        ),
        mesh=mesh,
        scratch_types=[
            pltpu.VMEM((N,), jnp.float32),
            pltpu.VMEM((N,), jnp.float32),
            pltpu.VMEM((N,), jnp.float32),
            pltpu.VMEM((QPW,), jnp.float32),
            pltpu.VMEM((QPW,), jnp.float32),
            pltpu.VMEM((QPW,), jnp.float32),
            pltpu.VMEM((S, QPW), jnp.int32),
            pltpu.VMEM((QPW,), jnp.int32),
            pltpu.VMEM((64,), jnp.int32),
        ],
    )(_ballq_body)
    idxT, idx_cnt = ballq(xyz_f, new_xyz_f)

    gath = functools.partial(
        pl.kernel,
        compiler_params=cparams,
        out_type=jax.ShapeDtypeStruct((B, CT, S, P), jnp.float32),
        mesh=mesh,
        scratch_types=[
            pltpu.VMEM((S, P), jnp.int32),
            pltpu.VMEM((N,), jnp.float32),
            pltpu.VMEM((P,), jnp.float32),
            pltpu.VMEM((1, 1, SSEG, P), jnp.float32),
        ],
    )(_gather_body)
    grouped = gath(xyz_f, new_xyz_f, feat_f, idxT)

    return idx_cnt.reshape(B, P), jnp.transpose(grouped, (0, 1, 3, 2))


# speedup vs baseline: 46.7288x; 2.9223x over previous
import functools

import jax
import jax.numpy as jnp
from jax import lax
from jax.experimental import pallas as pl
from jax.experimental.pallas import tpu as pltpu
from jax.experimental.pallas import tpu_sc as plsc

B, N, P, S, C = 4, 8192, 2048, 32, 256
CT = C + 3
R_IN2 = 0.8 * 0.8
R_OUT2 = 1.6 * 1.6
NC, NS, L = 2, 16, 16
NW = NC * NS
WPB = NW // B
QPW = P // WPB
SSEG = 8


def _worker_id():
    return lax.axis_index("s") * NC + lax.axis_index("c")


def _ballq_body(xyz_f, new_xyz_f, idx_out, cnt_out,
                xv, yv, zv, qx, qy, qz, idx_st, cnt_st, buf):
    w = _worker_id()
    b = w // WPB
    wi = w % WPB
    q0 = wi * QPW

    pltpu.sync_copy(xyz_f.at[pl.ds((b * 3 + 0) * N, N)], xv)
    pltpu.sync_copy(xyz_f.at[pl.ds((b * 3 + 1) * N, N)], yv)
    pltpu.sync_copy(xyz_f.at[pl.ds((b * 3 + 2) * N, N)], zv)
    pltpu.sync_copy(new_xyz_f.at[pl.ds((b * 3 + 0) * P + q0, QPW)], qx)
    pltpu.sync_copy(new_xyz_f.at[pl.ds((b * 3 + 1) * P + q0, QPW)], qy)
    pltpu.sync_copy(new_xyz_f.at[pl.ds((b * 3 + 2) * P + q0, QPW)], qz)

    iota = lax.iota(jnp.int32, L)
    lane0 = iota == 0

    def per_query(i, carry):
        fi = jnp.full((L,), i, jnp.int32)
        qxb = plsc.load_gather(qx, [fi])
        qyb = plsc.load_gather(qy, [fi])
        qzb = plsc.load_gather(qz, [fi])

        def cond(st):
            n, cnt = st
            return jnp.logical_and(cnt < S, n < N)

        def body(st):
            n, cnt = st
            dx = xv[pl.ds(n, L)] - qxb
            dy = yv[pl.ds(n, L)] - qyb
            dz = zv[pl.ds(n, L)] - qzb
            d2 = dx * dx + dy * dy + dz * dz
            m = jnp.logical_and(d2 >= R_IN2, d2 < R_OUT2)
            mi = m.astype(jnp.int32)
            tot = jnp.sum(mi)
            csum = plsc.cumsum(mi)
            pos = (cnt + csum) - mi
            plsc.store_scatter(buf, [pos], n + iota, mask=m)
            return n + L, cnt + tot

        _, cnt_end = lax.while_loop(cond, body, (jnp.int32(0), jnp.int32(0)))
        cntc = jnp.minimum(cnt_end, S)
        cntv = jnp.full((L,), cntc, jnp.int32)
        sv0 = buf[pl.ds(0, L)]
        h0 = jnp.where(cntc > 0, sv0[0], 0)
        first = jnp.full((L,), h0, jnp.int32)
        for j in range(S // L):
            sv = sv0 if j == 0 else buf[pl.ds(j * L, L)]
            outv = jnp.where(iota + (j * L) < cntv, sv, first)
            plsc.store_scatter(idx_st, [iota + (j * L), fi], outv)
        plsc.store_scatter(cnt_st, [fi], cntv, mask=lane0)
        return carry

    lax.fori_loop(0, QPW, per_query, 0)
    pltpu.sync_copy(idx_st, idx_out.at[pl.ds(b * S, S), pl.ds(q0, QPW)])
    pltpu.sync_copy(cnt_st, cnt_out.at[pl.ds(b * P + q0, QPW)])


def _gather_body(xyz_f, new_xyz_f, feat_f, idxT, out4,
                 idxv, src, qrow, obuf):
    w = _worker_id()
    b = w // WPB
    wi = w % WPB
    r0 = wi * 32 + jnp.minimum(wi, 3)
    nr = jnp.where(wi < 3, 33, 32)

    pltpu.sync_copy(idxT.at[pl.ds(b * S, S), :], idxv)

    def per_row(r, carry):
        cc = r0 + r

        def load_xyz():
            pltpu.sync_copy(xyz_f.at[pl.ds((b * 3 + cc) * N, N)], src)
            pltpu.sync_copy(new_xyz_f.at[pl.ds((b * 3 + cc) * P, P)], qrow)

        def load_feat():
            pltpu.sync_copy(feat_f.at[pl.ds((b * C + cc - 3) * N, N)], src)

        lax.cond(cc < 3, load_xyz, load_feat)

        def seg_loop(is_xyz):
            def per_seg(g, c2):
                def per_srow(sl, c3):
                    s = g * SSEG + sl

                    @plsc.parallel_loop(0, P, step=L, unroll=8)
                    def _pl(off):
                        iv = idxv[s, pl.ds(off, L)]
                        val = plsc.load_gather(src, [iv])
                        if is_xyz:
                            val = val - qrow[pl.ds(off, L)]
                        obuf[0, 0, sl, pl.ds(off, L)] = val
                    return c3

                lax.fori_loop(0, SSEG, per_srow, 0)
                pltpu.sync_copy(
                    obuf,
                    out4.at[pl.ds(b, 1), pl.ds(cc, 1), pl.ds(g * SSEG, SSEG), :])
                return c2

            lax.fori_loop(0, S // SSEG, per_seg, 0)

        lax.cond(cc < 3, lambda: seg_loop(True), lambda: seg_loop(False))
        return carry

    lax.fori_loop(0, nr, per_row, 0)


def kernel(xyz, new_xyz, features):
    xyz_f = jnp.transpose(xyz, (0, 2, 1)).reshape(-1)
    new_xyz_f = jnp.transpose(new_xyz, (0, 2, 1)).reshape(-1)
    feat_f = features.reshape(-1)

    mesh = plsc.VectorSubcoreMesh(
        core_axis_name="c", subcore_axis_name="s",
        num_cores=NC, num_subcores=NS)
    cparams = pltpu.CompilerParams(needs_layout_passes=False)

    ballq = functools.partial(
        pl.kernel,
        compiler_params=cparams,
        out_type=(
            jax.ShapeDtypeStruct((B * S, P), jnp.int32),
            jax.ShapeDtypeStruct((B * P,), jnp.int32),
        ),
        mesh=mesh,
        scratch_types=[
            pltpu.VMEM((N,), jnp.float32),
            pltpu.VMEM((N,), jnp.float32),
            pltpu.VMEM((N,), jnp.float32),
            pltpu.VMEM((QPW,), jnp.float32),
            pltpu.VMEM((QPW,), jnp.float32),
            pltpu.VMEM((QPW,), jnp.float32),
            pltpu.VMEM((S, QPW), jnp.int32),
            pltpu.VMEM((QPW,), jnp.int32),
            pltpu.VMEM((64,), jnp.int32),
        ],
    )(_ballq_body)
    idxT, idx_cnt = ballq(xyz_f, new_xyz_f)

    gath = functools.partial(
        pl.kernel,
        compiler_params=cparams,
        out_type=jax.ShapeDtypeStruct((B, CT, S, P), jnp.float32),
        mesh=mesh,
        scratch_types=[
            pltpu.VMEM((S, P), jnp.int32),
            pltpu.VMEM((N,), jnp.float32),
            pltpu.VMEM((P,), jnp.float32),
            pltpu.VMEM((1, 1, SSEG, P), jnp.float32),
        ],
    )(_gather_body)
    grouped = gath(xyz_f, new_xyz_f, feat_f, idxT)

    return idx_cnt.reshape(B, P), jnp.transpose(grouped, (0, 1, 3, 2))


# speedup vs baseline: 52.4448x; 1.1223x over previous
import functools

import jax
import jax.numpy as jnp
from jax import lax
from jax.experimental import pallas as pl
from jax.experimental.pallas import tpu as pltpu
from jax.experimental.pallas import tpu_sc as plsc

B, N, P, S, C = 4, 8192, 2048, 32, 256
CT = C + 3
R_IN2 = 0.8 * 0.8
R_OUT2 = 1.6 * 1.6
NC, NS, L = 2, 16, 16
NW = NC * NS
WPB = NW // B
QPW = P // WPB
SSEG = 8
PH = P // 2


def _worker_id():
    return lax.axis_index("s") * NC + lax.axis_index("c")


def _ballq_body(xyz_f, new_xyz_f, idx_out, cnt_out,
                xv, yv, zv, qx, qy, qz, idx_st, cnt_st, buf):
    w = _worker_id()
    b = w // WPB
    wi = w % WPB
    q0 = wi * QPW

    pltpu.sync_copy(xyz_f.at[pl.ds((b * 3 + 0) * N, N)], xv)
    pltpu.sync_copy(xyz_f.at[pl.ds((b * 3 + 1) * N, N)], yv)
    pltpu.sync_copy(xyz_f.at[pl.ds((b * 3 + 2) * N, N)], zv)
    pltpu.sync_copy(new_xyz_f.at[pl.ds((b * 3 + 0) * P + q0, QPW)], qx)
    pltpu.sync_copy(new_xyz_f.at[pl.ds((b * 3 + 1) * P + q0, QPW)], qy)
    pltpu.sync_copy(new_xyz_f.at[pl.ds((b * 3 + 2) * P + q0, QPW)], qz)

    iota = lax.iota(jnp.int32, L)
    lane0 = iota == 0

    def per_query(i, carry):
        fi = jnp.full((L,), i, jnp.int32)
        qxb = plsc.load_gather(qx, [fi])
        qyb = plsc.load_gather(qy, [fi])
        qzb = plsc.load_gather(qz, [fi])

        def cond(st):
            n, cnt = st
            return jnp.logical_and(cnt < S, n < N)

        def body(st):
            n, cnt = st
            dx = xv[pl.ds(n, L)] - qxb
            dy = yv[pl.ds(n, L)] - qyb
            dz = zv[pl.ds(n, L)] - qzb
            d2 = dx * dx + dy * dy + dz * dz
            m = jnp.logical_and(d2 >= R_IN2, d2 < R_OUT2)
            mi = m.astype(jnp.int32)
            tot = jnp.sum(mi)
            csum = plsc.cumsum(mi)
            pos = (cnt + csum) - mi
            plsc.store_scatter(buf, [pos], n + iota, mask=m)
            return n + L, cnt + tot

        _, cnt_end = lax.while_loop(cond, body, (jnp.int32(0), jnp.int32(0)))
        cntc = jnp.minimum(cnt_end, S)
        cntv = jnp.full((L,), cntc, jnp.int32)
        sv0 = buf[pl.ds(0, L)]
        h0 = jnp.where(cntc > 0, sv0[0], 0)
        first = jnp.full((L,), h0, jnp.int32)
        for j in range(S // L):
            sv = sv0 if j == 0 else buf[pl.ds(j * L, L)]
            outv = jnp.where(iota + (j * L) < cntv, sv, first)
            plsc.store_scatter(idx_st, [iota + (j * L), fi], outv)
        plsc.store_scatter(cnt_st, [fi], cntv, mask=lane0)
        return carry

    lax.fori_loop(0, QPW, per_query, 0)
    pltpu.sync_copy(idx_st, idx_out.at[pl.ds(b * S, S), pl.ds(q0, QPW)])
    pltpu.sync_copy(cnt_st, cnt_out.at[pl.ds(b * P + q0, QPW)])


def _gather_body(xyz_f, new_xyz_f, feat_f, idxT, out4,
                 idxv, src0, src1, qrow, ob0a, ob0b, ob1a, ob1b, sema, semb):
    w = _worker_id()
    b = w // WPB
    wi = w % WPB
    NSEG = S // SSEG

    for half in range(2):
        p0 = half * PH
        pltpu.sync_copy(idxT.at[pl.ds(b * S, S), pl.ds(p0, PH)], idxv)

        @pl.when(wi < 3)
        def _(p0=p0):
            cc = wi
            pltpu.sync_copy(xyz_f.at[pl.ds((b * 3 + cc) * N, N)], src0)
            pltpu.sync_copy(new_xyz_f.at[pl.ds((b * 3 + cc) * P + p0, PH)], qrow)

            def per_seg(g, c2):
                def per_srow(sl, c3):
                    s = g * SSEG + sl

                    @plsc.parallel_loop(0, PH, step=L, unroll=8)
                    def _pl(off):
                        iv = idxv[s, pl.ds(off, L)]
                        val = plsc.load_gather(src0, [iv]) - qrow[pl.ds(off, L)]
                        ob0a[0, 0, sl, pl.ds(off, L)] = val
                    return c3

                lax.fori_loop(0, SSEG, per_srow, 0)
                pltpu.sync_copy(
                    ob0a, out4.at[pl.ds(b, 1), pl.ds(cc, 1),
                                  pl.ds(g * SSEG, SSEG), pl.ds(p0, PH)])
                return c2

            lax.fori_loop(0, NSEG, per_seg, 0)

        def per_pair(pr, carry, p0=p0):
            f0 = wi * 32 + 2 * pr
            c0 = 3 + f0
            pltpu.sync_copy(feat_f.at[pl.ds((b * C + f0) * N, N)], src0)
            pltpu.sync_copy(feat_f.at[pl.ds((b * C + f0 + 1) * N, N)], src1)
            hs = {}
            for g in range(NSEG):
                par = g & 1
                ob0, ob1 = (ob0a, ob1a) if par == 0 else (ob0b, ob1b)
                sem = sema if par == 0 else semb
                if g >= 2:
                    hs[par][0].wait()
                    hs[par][1].wait()

                def per_srow(sl, c3, g=g, ob0=ob0, ob1=ob1):
                    s = g * SSEG + sl

                    @plsc.parallel_loop(0, PH, step=L, unroll=4)
                    def _pl(off):
                        iv = idxv[s, pl.ds(off, L)]
                        v0 = plsc.load_gather(src0, [iv])
                        v1 = plsc.load_gather(src1, [iv])
                        ob0[0, 0, sl, pl.ds(off, L)] = v0
                        ob1[0, 0, sl, pl.ds(off, L)] = v1
                    return c3

                lax.fori_loop(0, SSEG, per_srow, 0)
                h0 = pltpu.async_copy(
                    ob0, out4.at[pl.ds(b, 1), pl.ds(c0, 1),
                                 pl.ds(g * SSEG, SSEG), pl.ds(p0, PH)], sem)
                h1 = pltpu.async_copy(
                    ob1, out4.at[pl.ds(b, 1), pl.ds(c0 + 1, 1),
                                 pl.ds(g * SSEG, SSEG), pl.ds(p0, PH)], sem)
                hs[par] = (h0, h1)
            for par in range(2):
                hs[par][0].wait()
                hs[par][1].wait()
            return carry

        lax.fori_loop(0, C // (WPB * 2), per_pair, 0)


def kernel(xyz, new_xyz, features):
    xyz_f = jnp.transpose(xyz, (0, 2, 1)).reshape(-1)
    new_xyz_f = jnp.transpose(new_xyz, (0, 2, 1)).reshape(-1)
    feat_f = features.reshape(-1)

    mesh = plsc.VectorSubcoreMesh(
        core_axis_name="c", subcore_axis_name="s",
        num_cores=NC, num_subcores=NS)
    cparams = pltpu.CompilerParams(needs_layout_passes=False)

    ballq = functools.partial(
        pl.kernel,
        compiler_params=cparams,
        out_type=(
            jax.ShapeDtypeStruct((B * S, P), jnp.int32),
            jax.ShapeDtypeStruct((B * P,), jnp.int32),
        ),
        mesh=mesh,
        scratch_types=[
            pltpu.VMEM((N,), jnp.float32),
            pltpu.VMEM((N,), jnp.float32),
            pltpu.VMEM((N,), jnp.float32),
            pltpu.VMEM((QPW,), jnp.float32),
            pltpu.VMEM((QPW,), jnp.float32),
            pltpu.VMEM((QPW,), jnp.float32),
            pltpu.VMEM((S, QPW), jnp.int32),
            pltpu.VMEM((QPW,), jnp.int32),
            pltpu.VMEM((64,), jnp.int32),
        ],
    )(_ballq_body)
    idxT, idx_cnt = ballq(xyz_f, new_xyz_f)

    gath = functools.partial(
        pl.kernel,
        compiler_params=cparams,
        out_type=jax.ShapeDtypeStruct((B, CT, S, P), jnp.float32),
        mesh=mesh,
        scratch_types=[
            pltpu.VMEM((S, PH), jnp.int32),
            pltpu.VMEM((N,), jnp.float32),
            pltpu.VMEM((N,), jnp.float32),
            pltpu.VMEM((PH,), jnp.float32),
            pltpu.VMEM((1, 1, SSEG, PH), jnp.float32),
            pltpu.VMEM((1, 1, SSEG, PH), jnp.float32),
            pltpu.VMEM((1, 1, SSEG, PH), jnp.float32),
            pltpu.VMEM((1, 1, SSEG, PH), jnp.float32),
            pltpu.SemaphoreType.DMA,
            pltpu.SemaphoreType.DMA,
        ],
    )(_gather_body)
    grouped = gath(xyz_f, new_xyz_f, feat_f, idxT)

    return idx_cnt.reshape(B, P), jnp.transpose(grouped, (0, 1, 3, 2))


# speedup vs baseline: 56.9621x; 1.0861x over previous
import functools

import jax
import jax.numpy as jnp
from jax import lax
from jax.experimental import pallas as pl
from jax.experimental.pallas import tpu as pltpu
from jax.experimental.pallas import tpu_sc as plsc

B, N, P, S, C = 4, 8192, 2048, 32, 256
CT = C + 3
R_IN2 = 0.8 * 0.8
R_OUT2 = 1.6 * 1.6
NC, NS, L = 2, 16, 16
NW = NC * NS
WPB = NW // B
QPW = P // WPB
SSEG = 8
PH = P // 2


def _worker_id():
    return lax.axis_index("s") * NC + lax.axis_index("c")


def _ballq_body(xyz_f, new_xyz_f, idx_out, cnt_out,
                xv, yv, zv, qx, qy, qz, idx_st, cnt_st, buf):
    w = _worker_id()
    b = w // WPB
    wi = w % WPB
    q0 = wi * QPW

    pltpu.sync_copy(xyz_f.at[pl.ds((b * 3 + 0) * N, N)], xv)
    pltpu.sync_copy(xyz_f.at[pl.ds((b * 3 + 1) * N, N)], yv)
    pltpu.sync_copy(xyz_f.at[pl.ds((b * 3 + 2) * N, N)], zv)
    pltpu.sync_copy(new_xyz_f.at[pl.ds((b * 3 + 0) * P + q0, QPW)], qx)
    pltpu.sync_copy(new_xyz_f.at[pl.ds((b * 3 + 1) * P + q0, QPW)], qy)
    pltpu.sync_copy(new_xyz_f.at[pl.ds((b * 3 + 2) * P + q0, QPW)], qz)

    iota = lax.iota(jnp.int32, L)
    lane0 = iota == 0

    def per_query(i, carry):
        fi = jnp.full((L,), i, jnp.int32)
        qxb = plsc.load_gather(qx, [fi])
        qyb = plsc.load_gather(qy, [fi])
        qzb = plsc.load_gather(qz, [fi])

        def cond(st):
            n, cnt = st
            return jnp.logical_and(cnt < S, n < N)

        def body(st):
            n, cnt = st
            c = cnt
            for u in range(2):
                nn = n + u * L
                dx = xv[pl.ds(nn, L)] - qxb
                dy = yv[pl.ds(nn, L)] - qyb
                dz = zv[pl.ds(nn, L)] - qzb
                d2 = dx * dx + dy * dy + dz * dz
                m = jnp.logical_and(d2 >= R_IN2, d2 < R_OUT2)
                mi = m.astype(jnp.int32)
                csum = plsc.cumsum(mi)
                pos = (c + csum) - mi
                plsc.store_scatter(buf, [pos], nn + iota, mask=m)
                c = c + plsc.all_reduce_population_count(m)[0]
            return n + 2 * L, c

        _, cnt_end = lax.while_loop(cond, body, (jnp.int32(0), jnp.int32(0)))
        cntc = jnp.minimum(cnt_end, S)
        cntv = jnp.full((L,), cntc, jnp.int32)
        sv0 = buf[pl.ds(0, L)]
        h0 = jnp.where(cntc > 0, sv0[0], 0)
        first = jnp.full((L,), h0, jnp.int32)
        for j in range(S // L):
            sv = sv0 if j == 0 else buf[pl.ds(j * L, L)]
            outv = jnp.where(iota + (j * L) < cntv, sv, first)
            plsc.store_scatter(idx_st, [iota + (j * L), fi], outv)
        plsc.store_scatter(cnt_st, [fi], cntv, mask=lane0)
        return carry

    lax.fori_loop(0, QPW, per_query, 0)
    pltpu.sync_copy(idx_st, idx_out.at[pl.ds(b * S, S), pl.ds(q0, QPW)])
    pltpu.sync_copy(cnt_st, cnt_out.at[pl.ds(b * P + q0, QPW)])


def _gather_body(xyz_f, new_xyz_f, feat_f, idxT, out4,
                 idxv, src0, src1, qrow, ob0a, ob0b, ob1a, ob1b, sema, semb):
    w = _worker_id()
    b = w // WPB
    wi = w % WPB
    NSEG = S // SSEG

    for half in range(2):
        p0 = half * PH
        pltpu.sync_copy(idxT.at[pl.ds(b * S, S), pl.ds(p0, PH)], idxv)

        @pl.when(wi < 3)
        def _(p0=p0):
            cc = wi
            pltpu.sync_copy(xyz_f.at[pl.ds((b * 3 + cc) * N, N)], src0)
            pltpu.sync_copy(new_xyz_f.at[pl.ds((b * 3 + cc) * P + p0, PH)], qrow)

            def per_seg(g, c2):
                def per_srow(sl, c3):
                    s = g * SSEG + sl

                    @plsc.parallel_loop(0, PH, step=L, unroll=8)
                    def _pl(off):
                        iv = idxv[s, pl.ds(off, L)]
                        val = plsc.load_gather(src0, [iv]) - qrow[pl.ds(off, L)]
                        ob0a[0, 0, sl, pl.ds(off, L)] = val
                    return c3

                lax.fori_loop(0, SSEG, per_srow, 0)
                pltpu.sync_copy(
                    ob0a, out4.at[pl.ds(b, 1), pl.ds(cc, 1),
                                  pl.ds(g * SSEG, SSEG), pl.ds(p0, PH)])
                return c2

            lax.fori_loop(0, NSEG, per_seg, 0)

        def per_pair(pr, carry, p0=p0):
            f0 = wi * 32 + 2 * pr
            c0 = 3 + f0
            pltpu.sync_copy(feat_f.at[pl.ds((b * C + f0) * N, N)], src0)
            pltpu.sync_copy(feat_f.at[pl.ds((b * C + f0 + 1) * N, N)], src1)
            hs = {}
            for g in range(NSEG):
                par = g & 1
                ob0, ob1 = (ob0a, ob1a) if par == 0 else (ob0b, ob1b)
                sem = sema if par == 0 else semb
                if g >= 2:
                    hs[par][0].wait()
                    hs[par][1].wait()

                def per_srow(sl, c3, g=g, ob0=ob0, ob1=ob1):
                    s = g * SSEG + sl

                    @plsc.parallel_loop(0, PH, step=L, unroll=4)
                    def _pl(off):
                        iv = idxv[s, pl.ds(off, L)]
                        v0 = plsc.load_gather(src0, [iv])
                        v1 = plsc.load_gather(src1, [iv])
                        ob0[0, 0, sl, pl.ds(off, L)] = v0
                        ob1[0, 0, sl, pl.ds(off, L)] = v1
                    return c3

                lax.fori_loop(0, SSEG, per_srow, 0)
                h0 = pltpu.async_copy(
                    ob0, out4.at[pl.ds(b, 1), pl.ds(c0, 1),
                                 pl.ds(g * SSEG, SSEG), pl.ds(p0, PH)], sem)
                h1 = pltpu.async_copy(
                    ob1, out4.at[pl.ds(b, 1), pl.ds(c0 + 1, 1),
                                 pl.ds(g * SSEG, SSEG), pl.ds(p0, PH)], sem)
                hs[par] = (h0, h1)
            for par in range(2):
                hs[par][0].wait()
                hs[par][1].wait()
            return carry

        lax.fori_loop(0, C // (WPB * 2), per_pair, 0)


def kernel(xyz, new_xyz, features):
    xyz_f = jnp.transpose(xyz, (0, 2, 1)).reshape(-1)
    new_xyz_f = jnp.transpose(new_xyz, (0, 2, 1)).reshape(-1)
    feat_f = features.reshape(-1)

    mesh = plsc.VectorSubcoreMesh(
        core_axis_name="c", subcore_axis_name="s",
        num_cores=NC, num_subcores=NS)
    cparams = pltpu.CompilerParams(needs_layout_passes=False)

    ballq = functools.partial(
        pl.kernel,
        compiler_params=cparams,
        out_type=(
            jax.ShapeDtypeStruct((B * S, P), jnp.int32),
            jax.ShapeDtypeStruct((B * P,), jnp.int32),
        ),
        mesh=mesh,
        scratch_types=[
            pltpu.VMEM((N,), jnp.float32),
            pltpu.VMEM((N,), jnp.float32),
            pltpu.VMEM((N,), jnp.float32),
            pltpu.VMEM((QPW,), jnp.float32),
            pltpu.VMEM((QPW,), jnp.float32),
            pltpu.VMEM((QPW,), jnp.float32),
            pltpu.VMEM((S, QPW), jnp.int32),
            pltpu.VMEM((QPW,), jnp.int32),
            pltpu.VMEM((64,), jnp.int32),
        ],
    )(_ballq_body)
    idxT, idx_cnt = ballq(xyz_f, new_xyz_f)

    gath = functools.partial(
        pl.kernel,
        compiler_params=cparams,
        out_type=jax.ShapeDtypeStruct((B, CT, S, P), jnp.float32),
        mesh=mesh,
        scratch_types=[
            pltpu.VMEM((S, PH), jnp.int32),
            pltpu.VMEM((N,), jnp.float32),
            pltpu.VMEM((N,), jnp.float32),
            pltpu.VMEM((PH,), jnp.float32),
            pltpu.VMEM((1, 1, SSEG, PH), jnp.float32),
            pltpu.VMEM((1, 1, SSEG, PH), jnp.float32),
            pltpu.VMEM((1, 1, SSEG, PH), jnp.float32),
            pltpu.VMEM((1, 1, SSEG, PH), jnp.float32),
            pltpu.SemaphoreType.DMA,
            pltpu.SemaphoreType.DMA,
        ],
    )(_gather_body)
    grouped = gath(xyz_f, new_xyz_f, feat_f, idxT)

    return idx_cnt.reshape(B, P), jnp.transpose(grouped, (0, 1, 3, 2))


# speedup vs baseline: 72.0657x; 1.2652x over previous
import functools

import jax
import jax.numpy as jnp
from jax import lax
from jax.experimental import pallas as pl
from jax.experimental.pallas import tpu as pltpu
from jax.experimental.pallas import tpu_sc as plsc

B, N, P, S, C = 4, 8192, 2048, 32, 256
CT = C + 3
R_IN2 = 0.8 * 0.8
R_OUT2 = 1.6 * 1.6
NC, NS, L = 2, 16, 16
NW = NC * NS
WPB = NW // B
QPW = P // WPB
SSEG = 8
PH = P // 2
NPAIR = C // (WPB * 2)


def _worker_id():
    return lax.axis_index("s") * NC + lax.axis_index("c")


def _ballq_body(xyz_f, new_xyz_f, idx_out, cnt_out,
                xv, yv, zv, qx, qy, qz, idx_st, cnt_st, buf):
    w = _worker_id()
    b = w // WPB
    wi = w % WPB
    q0 = wi * QPW

    pltpu.sync_copy(xyz_f.at[pl.ds((b * 3 + 0) * N, N)], xv)
    pltpu.sync_copy(xyz_f.at[pl.ds((b * 3 + 1) * N, N)], yv)
    pltpu.sync_copy(xyz_f.at[pl.ds((b * 3 + 2) * N, N)], zv)
    pltpu.sync_copy(new_xyz_f.at[pl.ds((b * 3 + 0) * P + q0, QPW)], qx)
    pltpu.sync_copy(new_xyz_f.at[pl.ds((b * 3 + 1) * P + q0, QPW)], qy)
    pltpu.sync_copy(new_xyz_f.at[pl.ds((b * 3 + 2) * P + q0, QPW)], qz)

    iota = lax.iota(jnp.int32, L)
    lane0 = iota == 0

    def per_query(i, carry):
        fi = jnp.full((L,), i, jnp.int32)
        qxb = plsc.load_gather(qx, [fi])
        qyb = plsc.load_gather(qy, [fi])
        qzb = plsc.load_gather(qz, [fi])

        def cond(st):
            n, cnt = st
            return jnp.logical_and(cnt < S, n < N)

        def body(st):
            n, cnt = st
            c = cnt
            for u in range(4):
                nn = n + u * L
                dx = xv[pl.ds(nn, L)] - qxb
                dy = yv[pl.ds(nn, L)] - qyb
                dz = zv[pl.ds(nn, L)] - qzb
                d2 = dx * dx + dy * dy + dz * dz
                m = jnp.logical_and(d2 >= R_IN2, d2 < R_OUT2)
                mi = m.astype(jnp.int32)
                csum = plsc.cumsum(mi)
                pos = (c + csum) - mi
                plsc.store_scatter(buf, [pos], nn + iota, mask=m)
                c = c + plsc.all_reduce_population_count(m)[0]
            return n + 4 * L, c

        _, cnt_end = lax.while_loop(cond, body, (jnp.int32(0), jnp.int32(0)))
        cntc = jnp.minimum(cnt_end, S)
        cntv = jnp.full((L,), cntc, jnp.int32)
        sv0 = buf[pl.ds(0, L)]
        h0 = jnp.where(cntc > 0, sv0[0], 0)
        first = jnp.full((L,), h0, jnp.int32)
        for j in range(S // L):
            sv = sv0 if j == 0 else buf[pl.ds(j * L, L)]
            outv = jnp.where(iota + (j * L) < cntv, sv, first)
            plsc.store_scatter(idx_st, [iota + (j * L), fi], outv)
        plsc.store_scatter(cnt_st, [fi], cntv, mask=lane0)
        return carry

    lax.fori_loop(0, QPW, per_query, 0)
    pltpu.sync_copy(idx_st, idx_out.at[pl.ds(b * S, S), pl.ds(q0, QPW)])
    pltpu.sync_copy(cnt_st, cnt_out.at[pl.ds(b * P + q0, QPW)])


def _gather_body(xyz2, newxyz2, feat2, idxT, out4,
                 idxv, sA0, sA1, sB0, sB1, qrow,
                 ob0a, ob0b, ob1a, ob1b, sema, semb, semsA, semsB):
    w = _worker_id()
    b = w // WPB
    wi = w % WPB
    NSEG = S // SSEG

    def frow(f):
        r = jnp.minimum(b * C + f, B * C - 1)
        return feat2.at[pl.ds(r, 1), :]

    def fill_pair(s0, s1, ob0, ob1, g):
        def per_srow(sl, c3):
            s = g * SSEG + sl

            @plsc.parallel_loop(0, PH, step=L, unroll=4)
            def _pl(off):
                iv = idxv[s, pl.ds(off, L)]
                v0 = plsc.load_gather(s0.at[0], [iv])
                v1 = plsc.load_gather(s1.at[0], [iv])
                ob0[0, 0, sl, pl.ds(off, L)] = v0
                ob1[0, 0, sl, pl.ds(off, L)] = v1
            return c3

        lax.fori_loop(0, SSEG, per_srow, 0)

    for half in range(2):
        p0 = half * PH
        pltpu.sync_copy(idxT.at[pl.ds(b * S, S), pl.ds(p0, PH)], idxv)
        pltpu.async_copy(frow(wi * 32 + 0), sA0, semsA)
        pltpu.async_copy(frow(wi * 32 + 1), sA1, semsA)

        @pl.when(wi < 3)
        def _(p0=p0):
            cc = wi
            pltpu.sync_copy(xyz2.at[pl.ds(b * 3 + cc, 1), :], sB0)
            pltpu.sync_copy(newxyz2.at[pl.ds(b * 3 + cc, 1), pl.ds(p0, PH)], qrow)

            def per_seg(g, c2):
                def per_srow(sl, c3):
                    s = g * SSEG + sl

                    @plsc.parallel_loop(0, PH, step=L, unroll=8)
                    def _pl(off):
                        iv = idxv[s, pl.ds(off, L)]
                        val = plsc.load_gather(sB0.at[0], [iv]) - qrow[0, pl.ds(off, L)]
                        ob0a[0, 0, sl, pl.ds(off, L)] = val
                    return c3

                lax.fori_loop(0, SSEG, per_srow, 0)
                pltpu.sync_copy(
                    ob0a, out4.at[pl.ds(b, 1), pl.ds(cc, 1),
                                  pl.ds(g * SSEG, SSEG), pl.ds(p0, PH)])
                return c2

            lax.fori_loop(0, NSEG, per_seg, 0)

        def compute_pair(c0, s0, s1, p0):
            hs = {}
            for g in range(NSEG):
                par = g & 1
                ob0, ob1 = (ob0a, ob1a) if par == 0 else (ob0b, ob1b)
                sem = sema if par == 0 else semb
                if g >= 2:
                    hs[par][0].wait()
                    hs[par][1].wait()
                fill_pair(s0, s1, ob0, ob1, g)
                h0 = pltpu.async_copy(
                    ob0, out4.at[pl.ds(b, 1), pl.ds(c0, 1),
                                 pl.ds(g * SSEG, SSEG), pl.ds(p0, PH)], sem)
                h1 = pltpu.async_copy(
                    ob1, out4.at[pl.ds(b, 1), pl.ds(c0 + 1, 1),
                                 pl.ds(g * SSEG, SSEG), pl.ds(p0, PH)], sem)
                hs[par] = (h0, h1)
            for par in range(2):
                hs[par][0].wait()
                hs[par][1].wait()

        def per_pp(pp, carry, p0=p0):
            f0 = wi * 32 + 4 * pp
            pltpu.make_async_copy(frow(f0), sA0, semsA).wait()
            pltpu.make_async_copy(frow(f0 + 1), sA1, semsA).wait()
            pltpu.async_copy(frow(f0 + 2), sB0, semsB)
            pltpu.async_copy(frow(f0 + 3), sB1, semsB)
            compute_pair(3 + f0, sA0, sA1, p0)
            pltpu.make_async_copy(frow(f0 + 2), sB0, semsB).wait()
            pltpu.make_async_copy(frow(f0 + 3), sB1, semsB).wait()
            pltpu.async_copy(frow(f0 + 4), sA0, semsA)
            pltpu.async_copy(frow(f0 + 5), sA1, semsA)
            compute_pair(3 + f0 + 2, sB0, sB1, p0)
            return carry

        lax.fori_loop(0, NPAIR // 2, per_pp, 0)
        pltpu.make_async_copy(frow(wi * 32 + 32), sA0, semsA).wait()
        pltpu.make_async_copy(frow(wi * 32 + 33), sA1, semsA).wait()


def kernel(xyz, new_xyz, features):
    xyz_t = jnp.transpose(xyz, (0, 2, 1))
    new_xyz_t = jnp.transpose(new_xyz, (0, 2, 1))
    xyz_f = xyz_t.reshape(-1)
    new_xyz_f = new_xyz_t.reshape(-1)
    xyz2 = xyz_t.reshape(B * 3, N)
    newxyz2 = new_xyz_t.reshape(B * 3, P)
    feat2 = features.reshape(B * C, N)

    mesh = plsc.VectorSubcoreMesh(
        core_axis_name="c", subcore_axis_name="s",
        num_cores=NC, num_subcores=NS)
    cparams = pltpu.CompilerParams(needs_layout_passes=False)

    ballq = functools.partial(
        pl.kernel,
        compiler_params=cparams,
        out_type=(
            jax.ShapeDtypeStruct((B * S, P), jnp.int32),
            jax.ShapeDtypeStruct((B * P,), jnp.int32),
        ),
        mesh=mesh,
        scratch_types=[
            pltpu.VMEM((N,), jnp.float32),
            pltpu.VMEM((N,), jnp.float32),
            pltpu.VMEM((N,), jnp.float32),
            pltpu.VMEM((QPW,), jnp.float32),
            pltpu.VMEM((QPW,), jnp.float32),
            pltpu.VMEM((QPW,), jnp.float32),
            pltpu.VMEM((S, QPW), jnp.int32),
            pltpu.VMEM((QPW,), jnp.int32),
            pltpu.VMEM((128,), jnp.int32),
        ],
    )(_ballq_body)
    idxT, idx_cnt = ballq(xyz_f, new_xyz_f)

    gath = functools.partial(
        pl.kernel,
        compiler_params=cparams,
        out_type=jax.ShapeDtypeStruct((B, CT, S, P), jnp.float32),
        mesh=mesh,
        scratch_types=[
            pltpu.VMEM((S, PH), jnp.int32),
            pltpu.VMEM((1, N), jnp.float32),
            pltpu.VMEM((1, N), jnp.float32),
            pltpu.VMEM((1, N), jnp.float32),
            pltpu.VMEM((1, N), jnp.float32),
            pltpu.VMEM((1, PH), jnp.float32),
            pltpu.VMEM((1, 1, SSEG, PH), jnp.float32),
            pltpu.VMEM((1, 1, SSEG, PH), jnp.float32),
            pltpu.VMEM((1, 1, SSEG, PH), jnp.float32),
            pltpu.VMEM((1, 1, SSEG, PH), jnp.float32),
            pltpu.SemaphoreType.DMA,
            pltpu.SemaphoreType.DMA,
            pltpu.SemaphoreType.DMA,
            pltpu.SemaphoreType.DMA,
        ],
    )(_gather_body)
    grouped = gath(xyz2, newxyz2, feat2, idxT)

    return idx_cnt.reshape(B, P), jnp.transpose(grouped, (0, 1, 3, 2))


# speedup vs baseline: 72.7390x; 1.0093x over previous
import functools

import jax
import jax.numpy as jnp
from jax import lax
from jax.experimental import pallas as pl
from jax.experimental.pallas import tpu as pltpu
from jax.experimental.pallas import tpu_sc as plsc

B, N, P, S, C = 4, 8192, 2048, 32, 256
CT = C + 3
R_IN2 = 0.8 * 0.8
R_OUT2 = 1.6 * 1.6
NC, NS, L = 2, 16, 16
NW = NC * NS
WPB = NW // B
QPW = P // WPB
SSEG = 8
PH = P // 2
NPAIR = C // (WPB * 2)


def _fused_body(xyz2, newxyz2, feat2,
                idxT, cnt_out, out4,
                idxv, sA0, sA1, sB0, sB1, qrow,
                ob0a, ob0b, ob1a, ob1b, idx_st, cnt_st, buf,
                sema, semb, semsA, semsB):
    w = lax.axis_index("c") * NS + lax.axis_index("s")
    b = w // WPB
    wi = w % WPB
    q0 = wi * QPW
    NSEG = S // SSEG

    pltpu.sync_copy(xyz2.at[pl.ds(b * 3 + 0, 1), :], sA0)
    pltpu.sync_copy(xyz2.at[pl.ds(b * 3 + 1, 1), :], sA1)
    pltpu.sync_copy(xyz2.at[pl.ds(b * 3 + 2, 1), :], sB0)
    qxr = sB1.at[0, pl.ds(0 * QPW, QPW)]
    qyr = sB1.at[0, pl.ds(1 * QPW, QPW)]
    qzr = sB1.at[0, pl.ds(2 * QPW, QPW)]
    pltpu.sync_copy(newxyz2.at[pl.ds(b * 3 + 0, 1), pl.ds(q0, QPW)],
                    sB1.at[pl.ds(0, 1), pl.ds(0 * QPW, QPW)])
    pltpu.sync_copy(newxyz2.at[pl.ds(b * 3 + 1, 1), pl.ds(q0, QPW)],
                    sB1.at[pl.ds(0, 1), pl.ds(1 * QPW, QPW)])
    pltpu.sync_copy(newxyz2.at[pl.ds(b * 3 + 2, 1), pl.ds(q0, QPW)],
                    sB1.at[pl.ds(0, 1), pl.ds(2 * QPW, QPW)])

    iota = lax.iota(jnp.int32, L)
    lane0 = iota == 0

    def per_query(i, carry):
        fi = jnp.full((L,), i, jnp.int32)
        qxb = plsc.load_gather(qxr, [fi])
        qyb = plsc.load_gather(qyr, [fi])
        qzb = plsc.load_gather(qzr, [fi])

        def cond(st):
            n, cnt = st
            return jnp.logical_and(cnt < S, n < N)

        def body(st):
            n, cnt = st
            c = cnt
            for u in range(4):
                nn = n + u * L
                dx = sA0[0, pl.ds(nn, L)] - qxb
                dy = sA1[0, pl.ds(nn, L)] - qyb
                dz = sB0[0, pl.ds(nn, L)] - qzb
                d2 = dx * dx + dy * dy + dz * dz
                m = jnp.logical_and(d2 >= R_IN2, d2 < R_OUT2)
                mi = m.astype(jnp.int32)
                csum = plsc.cumsum(mi)
                pos = (c + csum) - mi
                plsc.store_scatter(buf, [pos], nn + iota, mask=m)
                c = c + plsc.all_reduce_population_count(m)[0]
            return n + 4 * L, c

        _, cnt_end = lax.while_loop(cond, body, (jnp.int32(0), jnp.int32(0)))
        cntc = jnp.minimum(cnt_end, S)
        cntv = jnp.full((L,), cntc, jnp.int32)
        sv0 = buf[pl.ds(0, L)]
        h0 = jnp.where(cntc > 0, sv0[0], 0)
        first = jnp.full((L,), h0, jnp.int32)
        for j in range(S // L):
            sv = sv0 if j == 0 else buf[pl.ds(j * L, L)]
            outv = jnp.where(iota + (j * L) < cntv, sv, first)
            plsc.store_scatter(idx_st, [iota + (j * L), fi], outv)
        plsc.store_scatter(cnt_st, [fi], cntv, mask=lane0)
        return carry

    lax.fori_loop(0, QPW, per_query, 0)
    pltpu.sync_copy(idx_st, idxT.at[pl.ds(b * S, S), pl.ds(q0, QPW)])
    pltpu.sync_copy(cnt_st, cnt_out.at[pl.ds(b * P + q0, QPW)])

    plsc.subcore_barrier()

    def frow(f):
        r = jnp.minimum(b * C + f, B * C - 1)
        return feat2.at[pl.ds(r, 1), :]

    def fill_pair(s0, s1, ob0, ob1, g):
        def per_srow(sl, c3):
            s = g * SSEG + sl

            @plsc.parallel_loop(0, PH, step=L, unroll=4)
            def _pl(off):
                iv = idxv[s, pl.ds(off, L)]
                v0 = plsc.load_gather(s0.at[0], [iv])
                v1 = plsc.load_gather(s1.at[0], [iv])
                ob0[0, 0, sl, pl.ds(off, L)] = v0
                ob1[0, 0, sl, pl.ds(off, L)] = v1
            return c3

        lax.fori_loop(0, SSEG, per_srow, 0)

    for half in range(2):
        p0 = half * PH
        pltpu.sync_copy(idxT.at[pl.ds(b * S, S), pl.ds(p0, PH)], idxv)
        pltpu.async_copy(frow(wi * 32 + 0), sA0, semsA)
        pltpu.async_copy(frow(wi * 32 + 1), sA1, semsA)

        @pl.when(wi < 3)
        def _(p0=p0):
            cc = wi
            pltpu.sync_copy(xyz2.at[pl.ds(b * 3 + cc, 1), :], sB0)
            pltpu.sync_copy(newxyz2.at[pl.ds(b * 3 + cc, 1), pl.ds(p0, PH)], qrow)

            def per_seg(g, c2):
                def per_srow(sl, c3):
                    s = g * SSEG + sl

                    @plsc.parallel_loop(0, PH, step=L, unroll=8)
                    def _pl(off):
                        iv = idxv[s, pl.ds(off, L)]
                        val = plsc.load_gather(sB0.at[0], [iv]) - qrow[0, pl.ds(off, L)]
                        ob0a[0, 0, sl, pl.ds(off, L)] = val
                    return c3

                lax.fori_loop(0, SSEG, per_srow, 0)
                pltpu.sync_copy(
                    ob0a, out4.at[pl.ds(b, 1), pl.ds(cc, 1),
                                  pl.ds(g * SSEG, SSEG), pl.ds(p0, PH)])
                return c2

            lax.fori_loop(0, NSEG, per_seg, 0)

        def compute_pair(c0, s0, s1, p0):
            hs = {}
            for g in range(NSEG):
                par = g & 1
                ob0, ob1 = (ob0a, ob1a) if par == 0 else (ob0b, ob1b)
                sem = sema if par == 0 else semb
                if g >= 2:
                    hs[par][0].wait()
                    hs[par][1].wait()
                fill_pair(s0, s1, ob0, ob1, g)
                h0 = pltpu.async_copy(
                    ob0, out4.at[pl.ds(b, 1), pl.ds(c0, 1),
                                 pl.ds(g * SSEG, SSEG), pl.ds(p0, PH)], sem)
                h1 = pltpu.async_copy(
                    ob1, out4.at[pl.ds(b, 1), pl.ds(c0 + 1, 1),
                                 pl.ds(g * SSEG, SSEG), pl.ds(p0, PH)], sem)
                hs[par] = (h0, h1)
            for par in range(2):
                hs[par][0].wait()
                hs[par][1].wait()

        def per_pp(pp, carry, p0=p0):
            f0 = wi * 32 + 4 * pp
            pltpu.make_async_copy(frow(f0), sA0, semsA).wait()
            pltpu.make_async_copy(frow(f0 + 1), sA1, semsA).wait()
            pltpu.async_copy(frow(f0 + 2), sB0, semsB)
            pltpu.async_copy(frow(f0 + 3), sB1, semsB)
            compute_pair(3 + f0, sA0, sA1, p0)
            pltpu.make_async_copy(frow(f0 + 2), sB0, semsB).wait()
            pltpu.make_async_copy(frow(f0 + 3), sB1, semsB).wait()
            pltpu.async_copy(frow(f0 + 4), sA0, semsA)
            pltpu.async_copy(frow(f0 + 5), sA1, semsA)
            compute_pair(3 + f0 + 2, sB0, sB1, p0)
            return carry

        lax.fori_loop(0, NPAIR // 2, per_pp, 0)
        pltpu.make_async_copy(frow(wi * 32 + 32), sA0, semsA).wait()
        pltpu.make_async_copy(frow(wi * 32 + 33), sA1, semsA).wait()


def kernel(xyz, new_xyz, features):
    xyz2 = jnp.transpose(xyz, (0, 2, 1)).reshape(B * 3, N)
    newxyz2 = jnp.transpose(new_xyz, (0, 2, 1)).reshape(B * 3, P)
    feat2 = features.reshape(B * C, N)

    mesh = plsc.VectorSubcoreMesh(
        core_axis_name="c", subcore_axis_name="s",
        num_cores=NC, num_subcores=NS)
    cparams = pltpu.CompilerParams(needs_layout_passes=False)

    fused = functools.partial(
        pl.kernel,
        compiler_params=cparams,
        out_type=(
            jax.ShapeDtypeStruct((B * S, P), jnp.int32),
            jax.ShapeDtypeStruct((B * P,), jnp.int32),
            jax.ShapeDtypeStruct((B, CT, S, P), jnp.float32),
        ),
        mesh=mesh,
        scratch_types=[
            pltpu.VMEM((S, PH), jnp.int32),
            pltpu.VMEM((1, N), jnp.float32),
            pltpu.VMEM((1, N), jnp.float32),
            pltpu.VMEM((1, N), jnp.float32),
            pltpu.VMEM((1, N), jnp.float32),
            pltpu.VMEM((1, PH), jnp.float32),
            pltpu.VMEM((1, 1, SSEG, PH), jnp.float32),
            pltpu.VMEM((1, 1, SSEG, PH), jnp.float32),
            pltpu.VMEM((1, 1, SSEG, PH), jnp.float32),
            pltpu.VMEM((1, 1, SSEG, PH), jnp.float32),
            pltpu.VMEM((S, QPW), jnp.int32),
            pltpu.VMEM((QPW,), jnp.int32),
            pltpu.VMEM((128,), jnp.int32),
            pltpu.SemaphoreType.DMA,
            pltpu.SemaphoreType.DMA,
            pltpu.SemaphoreType.DMA,
            pltpu.SemaphoreType.DMA,
        ],
    )(_fused_body)
    _, idx_cnt, grouped = fused(xyz2, newxyz2, feat2)

    return idx_cnt.reshape(B, P), jnp.transpose(grouped, (0, 1, 3, 2))


# speedup vs baseline: 73.3802x; 1.0088x over previous
import functools

import jax
import jax.numpy as jnp
from jax import lax
from jax.experimental import pallas as pl
from jax.experimental.pallas import tpu as pltpu
from jax.experimental.pallas import tpu_sc as plsc

B, N, P, S, C = 4, 8192, 2048, 32, 256
CT = C + 3
R_IN2 = 0.8 * 0.8
R_OUT2 = 1.6 * 1.6
NC, NS, L = 2, 16, 16
NW = NC * NS
WPB = NW // B
QPW = P // WPB
SSEG = 8
PH = P // 2
NPAIR = C // (WPB * 2)


def _fused_body(xyz2, newxyz2, feat2,
                idxT, cnt_out, out4,
                idxv, sA0, sA1, sB0, sB1, qrow,
                ob0a, ob0b, ob1a, ob1b, idx_st, cnt_st, buf,
                sema, semb, semsA, semsB):
    w = lax.axis_index("c") * NS + lax.axis_index("s")
    b = w // WPB
    wi = w % WPB
    q0 = wi * QPW
    NSEG = S // SSEG

    pltpu.sync_copy(xyz2.at[pl.ds(b * 3 + 0, 1), :], sA0)
    pltpu.sync_copy(xyz2.at[pl.ds(b * 3 + 1, 1), :], sA1)
    pltpu.sync_copy(xyz2.at[pl.ds(b * 3 + 2, 1), :], sB0)
    qxr = sB1.at[0, pl.ds(0 * QPW, QPW)]
    qyr = sB1.at[0, pl.ds(1 * QPW, QPW)]
    qzr = sB1.at[0, pl.ds(2 * QPW, QPW)]
    pltpu.sync_copy(newxyz2.at[pl.ds(b * 3 + 0, 1), pl.ds(q0, QPW)],
                    sB1.at[pl.ds(0, 1), pl.ds(0 * QPW, QPW)])
    pltpu.sync_copy(newxyz2.at[pl.ds(b * 3 + 1, 1), pl.ds(q0, QPW)],
                    sB1.at[pl.ds(0, 1), pl.ds(1 * QPW, QPW)])
    pltpu.sync_copy(newxyz2.at[pl.ds(b * 3 + 2, 1), pl.ds(q0, QPW)],
                    sB1.at[pl.ds(0, 1), pl.ds(2 * QPW, QPW)])

    iota = lax.iota(jnp.int32, L)
    lane0 = iota == 0

    def per_query(i, carry):
        fi = jnp.full((L,), i, jnp.int32)
        qxb = plsc.load_gather(qxr, [fi])
        qyb = plsc.load_gather(qyr, [fi])
        qzb = plsc.load_gather(qzr, [fi])

        def cond(st):
            n, cnt = st
            return jnp.logical_and(cnt < S, n < N)

        def body(st):
            n, cnt = st
            c = cnt
            for u in range(4):
                nn = n + u * L
                dx = sA0[0, pl.ds(nn, L)] - qxb
                dy = sA1[0, pl.ds(nn, L)] - qyb
                dz = sB0[0, pl.ds(nn, L)] - qzb
                d2 = dx * dx + dy * dy + dz * dz
                m = jnp.logical_and(d2 >= R_IN2, d2 < R_OUT2)
                mi = m.astype(jnp.int32)
                csum = plsc.cumsum(mi)
                pos = (c + csum) - mi
                plsc.store_scatter(buf, [pos], nn + iota, mask=m)
                c = c + plsc.all_reduce_population_count(m)[0]
            return n + 4 * L, c

        _, cnt_end = lax.while_loop(cond, body, (jnp.int32(0), jnp.int32(0)))
        cntc = jnp.minimum(cnt_end, S)
        cntv = jnp.full((L,), cntc, jnp.int32)
        sv0 = buf[pl.ds(0, L)]
        h0 = jnp.where(cntc > 0, sv0[0], 0)
        first = jnp.full((L,), h0, jnp.int32)
        for j in range(S // L):
            sv = sv0 if j == 0 else buf[pl.ds(j * L, L)]
            outv = jnp.where(iota + (j * L) < cntv, sv, first)
            plsc.store_scatter(idx_st, [iota + (j * L), fi], outv)
        plsc.store_scatter(cnt_st, [fi], cntv, mask=lane0)
        return carry

    lax.fori_loop(0, QPW, per_query, 0)
    pltpu.sync_copy(idx_st, idxT.at[pl.ds(b * S, S), pl.ds(q0, QPW)])
    pltpu.sync_copy(cnt_st, cnt_out.at[pl.ds(b * P + q0, QPW)])

    plsc.subcore_barrier()

    def frow(f):
        r = jnp.minimum(b * C + f, B * C - 1)
        return feat2.at[pl.ds(r, 1), :]

    def fill_pair(s0, s1, ob0, ob1, g):
        def per_srow(sl, c3):
            s = g * SSEG + sl

            @plsc.parallel_loop(0, PH, step=L, unroll=8)
            def _pl(off):
                iv = idxv[s, pl.ds(off, L)]
                v0 = plsc.load_gather(s0.at[0], [iv])
                v1 = plsc.load_gather(s1.at[0], [iv])
                ob0[0, 0, sl, pl.ds(off, L)] = v0
                ob1[0, 0, sl, pl.ds(off, L)] = v1
            return c3

        lax.fori_loop(0, SSEG, per_srow, 0)

    for half in range(2):
        p0 = half * PH
        pltpu.sync_copy(idxT.at[pl.ds(b * S, S), pl.ds(p0, PH)], idxv)
        pltpu.async_copy(frow(wi * 32 + 0), sA0, semsA)
        pltpu.async_copy(frow(wi * 32 + 1), sA1, semsA)

        @pl.when(wi < 3)
        def _(p0=p0):
            cc = wi
            pltpu.sync_copy(xyz2.at[pl.ds(b * 3 + cc, 1), :], sB0)
            pltpu.sync_copy(newxyz2.at[pl.ds(b * 3 + cc, 1), pl.ds(p0, PH)], qrow)

            def per_seg(g, c2):
                def per_srow(sl, c3):
                    s = g * SSEG + sl

                    @plsc.parallel_loop(0, PH, step=L, unroll=8)
                    def _pl(off):
                        iv = idxv[s, pl.ds(off, L)]
                        val = plsc.load_gather(sB0.at[0], [iv]) - qrow[0, pl.ds(off, L)]
                        ob0a[0, 0, sl, pl.ds(off, L)] = val
                    return c3

                lax.fori_loop(0, SSEG, per_srow, 0)
                pltpu.sync_copy(
                    ob0a, out4.at[pl.ds(b, 1), pl.ds(cc, 1),
                                  pl.ds(g * SSEG, SSEG), pl.ds(p0, PH)])
                return c2

            lax.fori_loop(0, NSEG, per_seg, 0)

        def compute_pair(c0, s0, s1, p0):
            hs = {}
            for g in range(NSEG):
                par = g & 1
                ob0, ob1 = (ob0a, ob1a) if par == 0 else (ob0b, ob1b)
                sem = sema if par == 0 else semb
                if g >= 2:
                    hs[par][0].wait()
                    hs[par][1].wait()
                fill_pair(s0, s1, ob0, ob1, g)
                h0 = pltpu.async_copy(
                    ob0, out4.at[pl.ds(b, 1), pl.ds(c0, 1),
                                 pl.ds(g * SSEG, SSEG), pl.ds(p0, PH)], sem)
                h1 = pltpu.async_copy(
                    ob1, out4.at[pl.ds(b, 1), pl.ds(c0 + 1, 1),
                                 pl.ds(g * SSEG, SSEG), pl.ds(p0, PH)], sem)
                hs[par] = (h0, h1)
            for par in range(2):
                hs[par][0].wait()
                hs[par][1].wait()

        def per_pp(pp, carry, p0=p0):
            f0 = wi * 32 + 4 * pp
            pltpu.make_async_copy(frow(f0), sA0, semsA).wait()
            pltpu.make_async_copy(frow(f0 + 1), sA1, semsA).wait()
            pltpu.async_copy(frow(f0 + 2), sB0, semsB)
            pltpu.async_copy(frow(f0 + 3), sB1, semsB)
            compute_pair(3 + f0, sA0, sA1, p0)
            pltpu.make_async_copy(frow(f0 + 2), sB0, semsB).wait()
            pltpu.make_async_copy(frow(f0 + 3), sB1, semsB).wait()
            pltpu.async_copy(frow(f0 + 4), sA0, semsA)
            pltpu.async_copy(frow(f0 + 5), sA1, semsA)
            compute_pair(3 + f0 + 2, sB0, sB1, p0)
            return carry

        lax.fori_loop(0, NPAIR // 2, per_pp, 0)
        pltpu.make_async_copy(frow(wi * 32 + 32), sA0, semsA).wait()
        pltpu.make_async_copy(frow(wi * 32 + 33), sA1, semsA).wait()


def kernel(xyz, new_xyz, features):
    xyz2 = jnp.transpose(xyz, (0, 2, 1)).reshape(B * 3, N)
    newxyz2 = jnp.transpose(new_xyz, (0, 2, 1)).reshape(B * 3, P)
    feat2 = features.reshape(B * C, N)

    mesh = plsc.VectorSubcoreMesh(
        core_axis_name="c", subcore_axis_name="s",
        num_cores=NC, num_subcores=NS)
    cparams = pltpu.CompilerParams(needs_layout_passes=False)

    fused = functools.partial(
        pl.kernel,
        compiler_params=cparams,
        out_type=(
            jax.ShapeDtypeStruct((B * S, P), jnp.int32),
            jax.ShapeDtypeStruct((B * P,), jnp.int32),
            jax.ShapeDtypeStruct((B, CT, S, P), jnp.float32),
        ),
        mesh=mesh,
        scratch_types=[
            pltpu.VMEM((S, PH), jnp.int32),
            pltpu.VMEM((1, N), jnp.float32),
            pltpu.VMEM((1, N), jnp.float32),
            pltpu.VMEM((1, N), jnp.float32),
            pltpu.VMEM((1, N), jnp.float32),
            pltpu.VMEM((1, PH), jnp.float32),
            pltpu.VMEM((1, 1, SSEG, PH), jnp.float32),
            pltpu.VMEM((1, 1, SSEG, PH), jnp.float32),
            pltpu.VMEM((1, 1, SSEG, PH), jnp.float32),
            pltpu.VMEM((1, 1, SSEG, PH), jnp.float32),
            pltpu.VMEM((S, QPW), jnp.int32),
            pltpu.VMEM((QPW,), jnp.int32),
            pltpu.VMEM((128,), jnp.int32),
            pltpu.SemaphoreType.DMA,
            pltpu.SemaphoreType.DMA,
            pltpu.SemaphoreType.DMA,
            pltpu.SemaphoreType.DMA,
        ],
    )(_fused_body)
    _, idx_cnt, grouped = fused(xyz2, newxyz2, feat2)

    return idx_cnt.reshape(B, P), jnp.transpose(grouped, (0, 1, 3, 2))


# speedup vs baseline: 77.7977x; 1.0602x over previous
import functools

import jax
import jax.numpy as jnp
from jax import lax
from jax.experimental import pallas as pl
from jax.experimental.pallas import tpu as pltpu
from jax.experimental.pallas import tpu_sc as plsc

B, N, P, S, C = 4, 8192, 2048, 32, 256
CT = C + 3
R_IN2 = 0.8 * 0.8
R_OUT2 = 1.6 * 1.6
NC, NS, L = 2, 16, 16
NW = NC * NS
WPB = NW // B
QPW = P // WPB
SSEG = 8
PH = P // 2
NPAIR = C // (WPB * 2)


def _fused_body(xyz2, newxyz2, feat2,
                idxT, cnt_out, out4,
                idxv, sA0, sA1, sB0, sB1, qrow,
                ob0a, ob0b, ob1a, ob1b, idx_st, idx16, cnt_st, buf,
                sema, semb, semsA, semsB):
    w = lax.axis_index("c") * NS + lax.axis_index("s")
    b = w // WPB
    wi = w % WPB
    q0 = wi * QPW
    NSEG = S // SSEG

    pltpu.sync_copy(xyz2.at[pl.ds(b * 3 + 0, 1), :], sA0)
    pltpu.sync_copy(xyz2.at[pl.ds(b * 3 + 1, 1), :], sA1)
    pltpu.sync_copy(xyz2.at[pl.ds(b * 3 + 2, 1), :], sB0)
    qxr = sB1.at[0, pl.ds(0 * QPW, QPW)]
    qyr = sB1.at[0, pl.ds(1 * QPW, QPW)]
    qzr = sB1.at[0, pl.ds(2 * QPW, QPW)]
    pltpu.sync_copy(newxyz2.at[pl.ds(b * 3 + 0, 1), pl.ds(q0, QPW)],
                    sB1.at[pl.ds(0, 1), pl.ds(0 * QPW, QPW)])
    pltpu.sync_copy(newxyz2.at[pl.ds(b * 3 + 1, 1), pl.ds(q0, QPW)],
                    sB1.at[pl.ds(0, 1), pl.ds(1 * QPW, QPW)])
    pltpu.sync_copy(newxyz2.at[pl.ds(b * 3 + 2, 1), pl.ds(q0, QPW)],
                    sB1.at[pl.ds(0, 1), pl.ds(2 * QPW, QPW)])

    iota = lax.iota(jnp.int32, L)
    lane0 = iota == 0

    def per_query(i, carry):
        fi = jnp.full((L,), i, jnp.int32)
        qxb = plsc.load_gather(qxr, [fi])
        qyb = plsc.load_gather(qyr, [fi])
        qzb = plsc.load_gather(qzr, [fi])

        def cond(st):
            n, cnt = st
            return jnp.logical_and(cnt < S, n < N)

        def body(st):
            n, cnt = st
            c = cnt
            for u in range(4):
                nn = n + u * L
                dx = sA0[0, pl.ds(nn, L)] - qxb
                dy = sA1[0, pl.ds(nn, L)] - qyb
                dz = sB0[0, pl.ds(nn, L)] - qzb
                d2 = dx * dx + dy * dy + dz * dz
                m = jnp.logical_and(d2 >= R_IN2, d2 < R_OUT2)
                mi = m.astype(jnp.int32)
                csum = plsc.cumsum(mi)
                pos = (c + csum) - mi
                plsc.store_scatter(buf, [pos], nn + iota, mask=m)
                c = c + plsc.all_reduce_population_count(m)[0]
            return n + 4 * L, c

        _, cnt_end = lax.while_loop(cond, body, (jnp.int32(0), jnp.int32(0)))
        cntc = jnp.minimum(cnt_end, S)
        cntv = jnp.full((L,), cntc, jnp.int32)
        sv0 = buf[pl.ds(0, L)]
        h0 = jnp.where(cntc > 0, sv0[0], 0)
        first = jnp.full((L,), h0, jnp.int32)
        for j in range(S // L):
            sv = sv0 if j == 0 else buf[pl.ds(j * L, L)]
            outv = jnp.where(iota + (j * L) < cntv, sv, first)
            plsc.store_scatter(idx_st, [iota + (j * L), fi], outv)
        plsc.store_scatter(cnt_st, [fi], cntv, mask=lane0)
        return carry

    lax.fori_loop(0, QPW, per_query, 0)

    def pack_row(s, c):
        def pack_k(k, c2):
            a = idx_st[s, pl.ds(k * 32, L)]
            bb = idx_st[s, pl.ds(k * 32 + L, L)]
            v = plsc.pack(a, bb, format=plsc.PackFormat.INTERLEAVED)
            idx16[s, pl.ds(k * 32, 2 * L)] = v
            return c2
        lax.fori_loop(0, QPW // (2 * L), pack_k, 0)
        return c

    lax.fori_loop(0, S, pack_row, 0)
    pltpu.sync_copy(idx16, idxT.at[pl.ds(b * S, S), pl.ds(q0, QPW)])
    pltpu.sync_copy(cnt_st, cnt_out.at[pl.ds(b * P + q0, QPW)])

    plsc.subcore_barrier()

    def frow(f):
        r = jnp.minimum(b * C + f, B * C - 1)
        return feat2.at[pl.ds(r, 1), :]

    def fill_pair(s0, s1, ob0, ob1, g):
        def per_srow(sl, c3):
            s = g * SSEG + sl

            @plsc.parallel_loop(0, PH, step=2 * L, unroll=4)
            def _pl(off):
                v16 = idxv[s, pl.ds(off, 2 * L)]
                iv0, iv1 = plsc.unpack(v16, format=plsc.PackFormat.INTERLEAVED)
                ob0[0, 0, sl, pl.ds(off, L)] = plsc.load_gather(s0.at[0], [iv0])
                ob1[0, 0, sl, pl.ds(off, L)] = plsc.load_gather(s1.at[0], [iv0])
                ob0[0, 0, sl, pl.ds(off + L, L)] = plsc.load_gather(s0.at[0], [iv1])
                ob1[0, 0, sl, pl.ds(off + L, L)] = plsc.load_gather(s1.at[0], [iv1])
            return c3

        lax.fori_loop(0, SSEG, per_srow, 0)

    for half in range(2):
        p0 = half * PH
        pltpu.sync_copy(idxT.at[pl.ds(b * S, S), pl.ds(p0, PH)], idxv)
        pltpu.async_copy(frow(wi * 32 + 0), sA0, semsA)
        pltpu.async_copy(frow(wi * 32 + 1), sA1, semsA)

        @pl.when(wi < 3)
        def _(p0=p0):
            cc = wi
            pltpu.sync_copy(xyz2.at[pl.ds(b * 3 + cc, 1), :], sB0)
            pltpu.sync_copy(newxyz2.at[pl.ds(b * 3 + cc, 1), pl.ds(p0, PH)], qrow)

            def per_seg(g, c2):
                def per_srow(sl, c3):
                    s = g * SSEG + sl

                    @plsc.parallel_loop(0, PH, step=2 * L, unroll=4)
                    def _pl(off):
                        v16 = idxv[s, pl.ds(off, 2 * L)]
                        iv0, iv1 = plsc.unpack(v16, format=plsc.PackFormat.INTERLEAVED)
                        ob0a[0, 0, sl, pl.ds(off, L)] = (
                            plsc.load_gather(sB0.at[0], [iv0]) - qrow[0, pl.ds(off, L)])
                        ob0a[0, 0, sl, pl.ds(off + L, L)] = (
                            plsc.load_gather(sB0.at[0], [iv1]) - qrow[0, pl.ds(off + L, L)])
                    return c3

                lax.fori_loop(0, SSEG, per_srow, 0)
                pltpu.sync_copy(
                    ob0a, out4.at[pl.ds(b, 1), pl.ds(cc, 1),
                                  pl.ds(g * SSEG, SSEG), pl.ds(p0, PH)])
                return c2

            lax.fori_loop(0, NSEG, per_seg, 0)

        def compute_pair(c0, s0, s1, p0):
            hs = {}
            for g in range(NSEG):
                par = g & 1
                ob0, ob1 = (ob0a, ob1a) if par == 0 else (ob0b, ob1b)
                sem = sema if par == 0 else semb
                if g >= 2:
                    hs[par][0].wait()
                    hs[par][1].wait()
                fill_pair(s0, s1, ob0, ob1, g)
                h0 = pltpu.async_copy(
                    ob0, out4.at[pl.ds(b, 1), pl.ds(c0, 1),
                                 pl.ds(g * SSEG, SSEG), pl.ds(p0, PH)], sem)
                h1 = pltpu.async_copy(
                    ob1, out4.at[pl.ds(b, 1), pl.ds(c0 + 1, 1),
                                 pl.ds(g * SSEG, SSEG), pl.ds(p0, PH)], sem)
                hs[par] = (h0, h1)
            for par in range(2):
                hs[par][0].wait()
                hs[par][1].wait()

        def per_pp(pp, carry, p0=p0):
            f0 = wi * 32 + 4 * pp
            pltpu.make_async_copy(frow(f0), sA0, semsA).wait()
            pltpu.make_async_copy(frow(f0 + 1), sA1, semsA).wait()
            pltpu.async_copy(frow(f0 + 2), sB0, semsB)
            pltpu.async_copy(frow(f0 + 3), sB1, semsB)
            compute_pair(3 + f0, sA0, sA1, p0)
            pltpu.make_async_copy(frow(f0 + 2), sB0, semsB).wait()
            pltpu.make_async_copy(frow(f0 + 3), sB1, semsB).wait()
            pltpu.async_copy(frow(f0 + 4), sA0, semsA)
            pltpu.async_copy(frow(f0 + 5), sA1, semsA)
            compute_pair(3 + f0 + 2, sB0, sB1, p0)
            return carry

        lax.fori_loop(0, NPAIR // 2, per_pp, 0)
        pltpu.make_async_copy(frow(wi * 32 + 32), sA0, semsA).wait()
        pltpu.make_async_copy(frow(wi * 32 + 33), sA1, semsA).wait()


def kernel(xyz, new_xyz, features):
    xyz2 = jnp.transpose(xyz, (0, 2, 1)).reshape(B * 3, N)
    newxyz2 = jnp.transpose(new_xyz, (0, 2, 1)).reshape(B * 3, P)
    feat2 = features.reshape(B * C, N)

    mesh = plsc.VectorSubcoreMesh(
        core_axis_name="c", subcore_axis_name="s",
        num_cores=NC, num_subcores=NS)
    cparams = pltpu.CompilerParams(needs_layout_passes=False)

    fused = functools.partial(
        pl.kernel,
        compiler_params=cparams,
        out_type=(
            jax.ShapeDtypeStruct((B * S, P), jnp.int16),
            jax.ShapeDtypeStruct((B * P,), jnp.int32),
            jax.ShapeDtypeStruct((B, CT, S, P), jnp.float32),
        ),
        mesh=mesh,
        scratch_types=[
            pltpu.VMEM((S, PH), jnp.int16),
            pltpu.VMEM((1, N), jnp.float32),
            pltpu.VMEM((1, N), jnp.float32),
            pltpu.VMEM((1, N), jnp.float32),
            pltpu.VMEM((1, N), jnp.float32),
            pltpu.VMEM((1, PH), jnp.float32),
            pltpu.VMEM((1, 1, SSEG, PH), jnp.float32),
            pltpu.VMEM((1, 1, SSEG, PH), jnp.float32),
            pltpu.VMEM((1, 1, SSEG, PH), jnp.float32),
            pltpu.VMEM((1, 1, SSEG, PH), jnp.float32),
            pltpu.VMEM((S, QPW), jnp.int32),
            pltpu.VMEM((S, QPW), jnp.int16),
            pltpu.VMEM((QPW,), jnp.int32),
            pltpu.VMEM((128,), jnp.int32),
            pltpu.SemaphoreType.DMA,
            pltpu.SemaphoreType.DMA,
            pltpu.SemaphoreType.DMA,
            pltpu.SemaphoreType.DMA,
        ],
    )(_fused_body)
    _, idx_cnt, grouped = fused(xyz2, newxyz2, feat2)

    return idx_cnt.reshape(B, P), jnp.transpose(grouped, (0, 1, 3, 2))
